# Initial kernel scaffold; baseline (speedup 1.0000x reference)
import jax
import jax.numpy as jnp
from jax.experimental import pallas as pl


def kernel(x, table):
    raise NotImplementedError("write your pallas kernel here")


# speedup vs baseline: 1.0062x; 1.0062x over previous
import functools

import jax
import jax.numpy as jnp
from jax import lax
from jax.experimental import pallas as pl
from jax.experimental.pallas import tpu as pltpu
from jax.experimental.pallas import tpu_sc as plsc

_NC = 2
_NS = 16
_L = 16


def _cascade_sc(x_flat, table_flat, B, SL):
    NW = _NC * _NS
    flat = (B // NW) * SL
    groups = B // NW // _L

    mesh = plsc.VectorSubcoreMesh(core_axis_name="c", subcore_axis_name="s")

    @functools.partial(
        pl.kernel,
        mesh=mesh,
        out_type=jax.ShapeDtypeStruct((B * SL,), jnp.float32),
        scratch_types=[
            pltpu.VMEM((flat,), jnp.int32),
            pltpu.VMEM((flat,), jnp.float32),
            pltpu.VMEM((flat,), jnp.float32),
            pltpu.SemaphoreType.DMA,
        ],
        compiler_params=pltpu.CompilerParams(needs_layout_passes=False),
    )
    def run(x_hbm, table_hbm, out_hbm, idx_v, vals_v, out_v, sem):
        wid = lax.axis_index("s") * _NC + lax.axis_index("c")
        base = wid * flat
        pltpu.sync_copy(x_hbm.at[pl.ds(base, flat)], idx_v)
        pltpu.async_copy(table_hbm.at[idx_v], vals_v, sem).wait()

        lane = lax.iota(jnp.int32, _L) * SL

        def row_group(j, _):
            gbase = j * (_L * SL)

            def step(l, cum):
                iv = lane + (gbase + l)
                v = plsc.load_gather(vals_v, [iv])
                r = 1.0 / (1.0 + jnp.exp(-v))
                plsc.store_scatter(out_v, [iv], cum * r)
                return cum * (1.0 - r)

            lax.fori_loop(0, SL, step, jnp.full((_L,), 1.0, jnp.float32))
            return 0

        lax.fori_loop(0, groups, row_group, 0)
        pltpu.sync_copy(out_v, out_hbm.at[pl.ds(base, flat)])

    return run(x_flat, table_flat)


def kernel(x, table):
    B, SL = x.shape
    out = _cascade_sc(x.reshape(-1), table.reshape(-1), B, SL)
    return out.reshape(B, SL)


# speedup vs baseline: 1.0515x; 1.0451x over previous
import functools

import jax
import jax.numpy as jnp
from jax import lax
from jax.experimental import pallas as pl
from jax.experimental.pallas import tpu as pltpu
from jax.experimental.pallas import tpu_sc as plsc

_NC = 2
_NS = 16
_L = 16
_CH = 8


def _cascade_sc(x_flat, table_flat, B, SL):
    NW = _NC * _NS
    flat = (B // NW) * SL
    chunk = flat // _CH
    gpc = chunk // (_L * SL)

    mesh = plsc.VectorSubcoreMesh(core_axis_name="c", subcore_axis_name="s")

    @functools.partial(
        pl.kernel,
        mesh=mesh,
        out_type=jax.ShapeDtypeStruct((B * SL,), jnp.float32),
        scratch_types=[
            pltpu.VMEM((flat,), jnp.int32),
            pltpu.VMEM((chunk,), jnp.float32),
            pltpu.VMEM((chunk,), jnp.float32),
            pltpu.VMEM((flat,), jnp.float32),
            pltpu.SemaphoreType.DMA,
            pltpu.SemaphoreType.DMA,
            pltpu.SemaphoreType.DMA,
        ],
        compiler_params=pltpu.CompilerParams(needs_layout_passes=False),
    )
    def run(x_hbm, table_hbm, out_hbm, idx_v, vals0, vals1, out_v,
            gsem0, gsem1, osem):
        wid = lax.axis_index("s") * _NC + lax.axis_index("c")
        base = wid * flat
        pltpu.sync_copy(x_hbm.at[pl.ds(base, flat)], idx_v)

        bufs = (vals0, vals1)
        sems = (gsem0, gsem1)
        lane = lax.iota(jnp.int32, _L) * SL

        def scan_chunk(k, buf):
            obase = k * chunk

            def group(g, _):
                gbase = g * (_L * SL)
                cum = jnp.full((_L,), 1.0, jnp.float32)
                for l in range(SL):
                    iv = lane + (gbase + l)
                    v = plsc.load_gather(buf, [iv])
                    r = 1.0 / (1.0 + jnp.exp(-v))
                    plsc.store_scatter(out_v, [obase + iv], cum * r)
                    cum = cum * (1.0 - r)
                return 0

            lax.fori_loop(0, gpc, group, 0)

        gath = [None] * _CH
        gath[0] = pltpu.async_copy(
            table_hbm.at[idx_v.at[pl.ds(0, chunk)]], bufs[0], sems[0])
        outs = []
        for k in range(_CH):
            if k + 1 < _CH:
                gath[k + 1] = pltpu.async_copy(
                    table_hbm.at[idx_v.at[pl.ds((k + 1) * chunk, chunk)]],
                    bufs[(k + 1) % 2], sems[(k + 1) % 2])
            gath[k].wait()
            scan_chunk(k, bufs[k % 2])
            outs.append(pltpu.async_copy(
                out_v.at[pl.ds(k * chunk, chunk)],
                out_hbm.at[pl.ds(base + k * chunk, chunk)], osem))
        for h in outs:
            h.wait()

    return run(x_flat, table_flat)


def kernel(x, table):
    B, SL = x.shape
    out = _cascade_sc(x.reshape(-1), table.reshape(-1), B, SL)
    return out.reshape(B, SL)


# speedup vs baseline: 1.1107x; 1.0563x over previous
import functools

import jax
import jax.numpy as jnp
from jax import lax
from jax.experimental import pallas as pl
from jax.experimental.pallas import tpu as pltpu
from jax.experimental.pallas import tpu_sc as plsc

_NC = 2
_NS = 16
_L = 16
_CH = 8


def _cascade_sc(x_flat, table_flat, B, SL):
    NW = _NC * _NS
    flat = (B // NW) * SL
    chunk = flat // _CH
    gpc = chunk // (_L * SL)

    mesh = plsc.VectorSubcoreMesh(core_axis_name="c", subcore_axis_name="s")

    @functools.partial(
        pl.kernel,
        mesh=mesh,
        out_type=jax.ShapeDtypeStruct((B * SL,), jnp.float32),
        scratch_types=[
            pltpu.VMEM((flat,), jnp.int32),
            pltpu.VMEM((chunk,), jnp.float32),
            pltpu.VMEM((chunk,), jnp.float32),
            pltpu.VMEM((flat,), jnp.float32),
            pltpu.SemaphoreType.DMA,
            pltpu.SemaphoreType.DMA,
            pltpu.SemaphoreType.DMA,
            pltpu.SemaphoreType.DMA,
        ],
        compiler_params=pltpu.CompilerParams(needs_layout_passes=False),
    )
    def run(x_hbm, table_hbm, out_hbm, idx_v, vals0, vals1, out_v,
            gsem0, gsem1, osem, isem):
        wid = lax.axis_index("s") * _NC + lax.axis_index("c")
        base = wid * flat

        bufs = (vals0, vals1)
        sems = (gsem0, gsem1)
        lane = lax.iota(jnp.int32, _L) * SL

        def scan_chunk(k, buf):
            obase = k * chunk

            def group(g, _):
                gbase = g * (_L * SL)
                ivs = [lane + (gbase + l) for l in range(SL)]
                vs = [plsc.load_gather(buf, [iv]) for iv in ivs]
                rs = [1.0 / (1.0 + jnp.exp(-v)) for v in vs]
                cum = jnp.full((_L,), 1.0, jnp.float32)
                for l in range(SL):
                    plsc.store_scatter(out_v, [obase + ivs[l]], cum * rs[l])
                    cum = cum * (1.0 - rs[l])
                return 0

            lax.fori_loop(0, gpc, group, 0)

        def idx_copy(k):
            return pltpu.async_copy(
                x_hbm.at[pl.ds(base + k * chunk, chunk)],
                idx_v.at[pl.ds(k * chunk, chunk)], isem)

        def gather(k):
            return pltpu.async_copy(
                table_hbm.at[idx_v.at[pl.ds(k * chunk, chunk)]],
                bufs[k % 2], sems[k % 2])

        ih = [idx_copy(k) for k in range(_CH)]
        ih[0].wait()
        gath = [None] * _CH
        gath[0] = gather(0)
        outs = []
        for k in range(_CH):
            if k + 1 < _CH:
                ih[k + 1].wait()
                gath[k + 1] = gather(k + 1)
            gath[k].wait()
            scan_chunk(k, bufs[k % 2])
            outs.append(pltpu.async_copy(
                out_v.at[pl.ds(k * chunk, chunk)],
                out_hbm.at[pl.ds(base + k * chunk, chunk)], osem))
        for h in outs:
            h.wait()

    return run(x_flat, table_flat)


def kernel(x, table):
    B, SL = x.shape
    out = _cascade_sc(x.reshape(-1), table.reshape(-1), B, SL)
    return out.reshape(B, SL)


# speedup vs baseline: 1.1121x; 1.0013x over previous
import functools

import jax
import jax.numpy as jnp
from jax import lax
from jax.experimental import pallas as pl
from jax.experimental.pallas import tpu as pltpu
from jax.experimental.pallas import tpu_sc as plsc

_NC = 2
_NS = 16
_L = 16
_CH = 8


def _cascade_sc(x_flat, table_flat, B, SL):
    NW = _NC * _NS
    flat = (B // NW) * SL
    chunk = flat // _CH
    gpc = chunk // (_L * SL)

    mesh = plsc.VectorSubcoreMesh(core_axis_name="c", subcore_axis_name="s")

    @functools.partial(
        pl.kernel,
        mesh=mesh,
        out_type=jax.ShapeDtypeStruct((B * SL,), jnp.float32),
        scratch_types=[
            pltpu.VMEM((flat,), jnp.int32),
            pltpu.VMEM((chunk,), jnp.float32),
            pltpu.VMEM((chunk,), jnp.float32),
            pltpu.VMEM((flat,), jnp.float32),
            pltpu.SemaphoreType.DMA,
            pltpu.SemaphoreType.DMA,
            pltpu.SemaphoreType.DMA,
            pltpu.SemaphoreType.DMA,
        ],
        compiler_params=pltpu.CompilerParams(
            needs_layout_passes=False,
            skip_device_barrier=True,
            disable_bounds_checks=True,
        ),
    )
    def run(x_hbm, table_hbm, out_hbm, idx_v, vals0, vals1, out_v,
            gsem0, gsem1, osem, isem):
        wid = lax.axis_index("s") * _NC + lax.axis_index("c")
        base = wid * flat

        bufs = (vals0, vals1)
        sems = (gsem0, gsem1)
        lane = lax.iota(jnp.int32, _L) * SL

        def scan_chunk(k, buf):
            obase = k * chunk

            def group(g, _):
                gbase = g * (_L * SL)
                ivs = [lane + (gbase + l) for l in range(SL)]
                vs = [plsc.load_gather(buf, [iv]) for iv in ivs]
                rs = [1.0 / (1.0 + jnp.exp(-v)) for v in vs]
                cum = jnp.full((_L,), 1.0, jnp.float32)
                for l in range(SL):
                    plsc.store_scatter(out_v, [obase + ivs[l]], cum * rs[l])
                    cum = cum * (1.0 - rs[l])
                return 0

            lax.fori_loop(0, gpc, group, 0)

        def idx_copy(k):
            return pltpu.async_copy(
                x_hbm.at[pl.ds(base + k * chunk, chunk)],
                idx_v.at[pl.ds(k * chunk, chunk)], isem)

        def gather(k):
            return pltpu.async_copy(
                table_hbm.at[idx_v.at[pl.ds(k * chunk, chunk)]],
                bufs[k % 2], sems[k % 2])

        ih = [idx_copy(k) for k in range(_CH)]
        ih[0].wait()
        gath = [None] * _CH
        gath[0] = gather(0)
        outs = []
        for k in range(_CH):
            if k + 1 < _CH:
                ih[k + 1].wait()
                gath[k + 1] = gather(k + 1)
            gath[k].wait()
            scan_chunk(k, bufs[k % 2])
            outs.append(pltpu.async_copy(
                out_v.at[pl.ds(k * chunk, chunk)],
                out_hbm.at[pl.ds(base + k * chunk, chunk)], osem))
        for h in outs:
            h.wait()

    return run(x_flat, table_flat)


def kernel(x, table):
    B, SL = x.shape
    out = _cascade_sc(x.reshape(-1), table.reshape(-1), B, SL)
    return out.reshape(B, SL)
